# baseline (speedup 1.0000x reference)
import functools

import jax
import jax.numpy as jnp
from jax import lax
from jax.experimental import pallas as pl
from jax.experimental.pallas import tpu as pltpu
from jax.experimental.pallas import tpu_sc as plsc

B = 16384
D_BATCH, D_DONOR, D_ASSAY = 64, 64, 32
D_OUT = D_BATCH + D_DONOR + D_ASSAY
V_SMALL = 1000
V_TILES = V_SMALL // 8

NC, NS = 2, 16
NW = NC * NS
BPW = B // NW
CHUNK = 128
NCH = BPW // CHUNK

_mesh = plsc.VectorSubcoreMesh(core_axis_name="c", subcore_axis_name="s")


@functools.partial(
    pl.kernel,
    out_type=jax.ShapeDtypeStruct((D_OUT, B), jnp.float32),
    mesh=_mesh,
    scratch_types=[
        pltpu.VMEM((BPW,), jnp.int32),
        pltpu.VMEM((BPW,), jnp.int32),
        pltpu.VMEM((BPW,), jnp.int32),
        pltpu.VMEM((BPW,), jnp.int32),
        pltpu.VMEM((D_OUT, CHUNK), jnp.float32),
        pltpu.VMEM((CHUNK, 128), jnp.float32),
        pltpu.VMEM((CHUNK, 128), jnp.float32),
        pltpu.VMEM((CHUNK, 128), jnp.float32),
        pltpu.VMEM((8, D_BATCH), jnp.float32),
        pltpu.VMEM((8, D_ASSAY), jnp.float32),
        pltpu.VMEM((8, 128), jnp.float32),
        pltpu.VMEM_SHARED((V_SMALL, 128), jnp.float32),
        pltpu.SemaphoreType.DMA,
        pltpu.SemaphoreType.DMA,
        pltpu.SemaphoreType.DMA,
    ],
    compiler_params=pltpu.CompilerParams(needs_layout_passes=False),
)
def _embed_concat(b_idx, d_idx, a_idx, Wb, Wd2, Wa, out_t,
                  ib, idn, ia, idn2, bigT, rdc, rbc, rac, tb, ta, vb, spc,
                  sem_i, sem_g, sem_s):
    sid = lax.axis_index("s")
    wid = lax.axis_index("c") * NS + sid
    base = wid * BPW
    lane = lax.broadcasted_iota(jnp.int32, (16,), 0)
    iload = [
        pltpu.async_copy(b_idx.at[pl.ds(base, BPW)], ib, sem_i),
        pltpu.async_copy(d_idx.at[pl.ds(base, BPW)], idn, sem_i),
        pltpu.async_copy(a_idx.at[pl.ds(base, BPW)], ia, sem_i),
    ]

    @pl.loop(0, (V_TILES + NS - 1) // NS)
    def stage_loop(t):
        k = t * NS + sid

        @pl.when(k < V_TILES)
        def _():
            off = k * 8
            pltpu.sync_copy(Wb.at[pl.ds(off, 8)], tb)
            pltpu.sync_copy(Wa.at[pl.ds(off, 8)], ta)
            for r in range(8):
                for j in range(4):
                    vb[r, pl.ds(j * 16, 16)] = tb[r, pl.ds(j * 16, 16)]
                for j in range(2):
                    vb[r, pl.ds(D_BATCH + j * 16, 16)] = ta[r, pl.ds(j * 16, 16)]
            pltpu.sync_copy(vb, spc.at[pl.ds(off, 8)])

    for c in iload:
        c.wait()

    @pl.loop(0, BPW // 16)
    def mkidx(t):
        idn2[pl.ds(t * 16, 16)] = idn[pl.ds(t * 16, 16)] // 2

    plsc.subcore_barrier()

    @pl.loop(0, NCH)
    def chunk_loop(ch):
        cbase = ch * CHUNK
        gd = pltpu.async_copy(Wd2.at[idn2.at[pl.ds(cbase, CHUNK)]], rdc, sem_s)
        gb = pltpu.async_copy(spc.at[ib.at[pl.ds(cbase, CHUNK)]], rbc, sem_g)
        ga = pltpu.async_copy(spc.at[ia.at[pl.ds(cbase, CHUNK)]], rac, sem_g)
        gd.wait()
        gb.wait()
        ga.wait()

        for h in range(CHUNK // 16):
            ivd = idn[pl.ds(cbase + h * 16, 16)]
            for r in range(16):
                row = h * 16 + r
                half = ivd[r] - (ivd[r] // 2) * 2
                hoff = half * D_DONOR
                col = jnp.full((16,), row, jnp.int32)
                for j in range(4):
                    v = rdc[row, pl.ds(hoff + j * 16, 16)]
                    plsc.store_scatter(
                        bigT, [D_BATCH + j * 16 + lane, col], v)
                for j in range(4):
                    v = rbc[row, pl.ds(j * 16, 16)]
                    plsc.store_scatter(bigT, [j * 16 + lane, col], v)
                for j in range(2):
                    v = rac[row, pl.ds(D_BATCH + j * 16, 16)]
                    plsc.store_scatter(
                        bigT, [D_BATCH + D_DONOR + j * 16 + lane, col], v)

        pltpu.sync_copy(bigT, out_t.at[:, pl.ds(base + cbase, CHUNK)])


def kernel(batch, donor, assay, W_batch, W_donor, W_assay):
    b1 = batch.astype(jnp.int32)
    d1 = donor.astype(jnp.int32)
    a1 = assay.astype(jnp.int32)
    wd2 = jnp.reshape(W_donor, (W_donor.shape[0] // 2, 128))
    ot = _embed_concat(b1, d1, a1, W_batch, wd2, W_assay)
    return jnp.transpose(ot)


# speedup vs baseline: 1.2877x; 1.2877x over previous
import functools

import jax
import jax.numpy as jnp
from jax import lax
from jax.experimental import pallas as pl
from jax.experimental.pallas import tpu as pltpu
from jax.experimental.pallas import tpu_sc as plsc

B = 16384
D_BATCH, D_DONOR, D_ASSAY = 64, 64, 32
D_OUT = D_BATCH + D_DONOR + D_ASSAY

NC, NS = 2, 16
NW = NC * NS
BPW = B // NW

_mesh = plsc.VectorSubcoreMesh(core_axis_name="c", subcore_axis_name="s")


@functools.partial(
    pl.kernel,
    out_type=jax.ShapeDtypeStruct((B, D_OUT), jnp.float32),
    mesh=_mesh,
    scratch_types=[
        pltpu.VMEM((BPW,), jnp.int32),
        pltpu.VMEM((BPW,), jnp.int32),
        pltpu.VMEM((BPW,), jnp.int32),
        pltpu.VMEM((BPW, D_BATCH), jnp.float32),
        pltpu.VMEM((BPW, D_DONOR), jnp.float32),
        pltpu.VMEM((BPW, D_ASSAY), jnp.float32),
        pltpu.SemaphoreType.DMA,
    ],
    compiler_params=pltpu.CompilerParams(use_tc_tiling_on_sc=False),
)
def _embed_concat(b_idx, d_idx, a_idx, Wb, Wd, Wa, out,
                  ib, idn, ia, rb, rd, ra, sem):
    wid = lax.axis_index("c") * NS + lax.axis_index("s")
    base = wid * BPW
    rows = pl.ds(base, BPW)
    iload = [
        pltpu.async_copy(b_idx.at[rows], ib, sem),
        pltpu.async_copy(d_idx.at[rows], idn, sem),
        pltpu.async_copy(a_idx.at[rows], ia, sem),
    ]
    for c in iload:
        c.wait()
    gathers = [
        pltpu.async_copy(Wb.at[ib], rb, sem),
        pltpu.async_copy(Wd.at[idn], rd, sem),
        pltpu.async_copy(Wa.at[ia], ra, sem),
    ]
    for c in gathers:
        c.wait()
    pltpu.sync_copy(rb, out.at[rows, pl.ds(0, D_BATCH)])
    pltpu.sync_copy(rd, out.at[rows, pl.ds(D_BATCH, D_DONOR)])
    pltpu.sync_copy(ra, out.at[rows, pl.ds(D_BATCH + D_DONOR, D_ASSAY)])


def kernel(batch, donor, assay, W_batch, W_donor, W_assay):
    b1 = batch.astype(jnp.int32)
    d1 = donor.astype(jnp.int32)
    a1 = assay.astype(jnp.int32)
    return _embed_concat(b1, d1, a1, W_batch, W_donor, W_assay)
